# Initial kernel scaffold; baseline (speedup 1.0000x reference)
import jax
import jax.numpy as jnp
from jax.experimental import pallas as pl


def kernel(inputs, edges, hidden, W_ef1, b_ef1, W_ef2, b_ef2, W_res, b_res, W_o1, b_o1, W_o2, b_o2, W_o3, b_o3):
    raise NotImplementedError("write your pallas kernel here")


# speedup vs baseline: 16.2308x; 16.2308x over previous
import functools

import jax
import jax.numpy as jnp
import numpy as np
from jax import lax
from jax.experimental import pallas as pl

B, N, E, H, IN = 4, 512, 261632, 64, 4
K = 1024
EP = 262144
NB = EP // K
F = 8
TWO_PI = np.float32(2.0 * np.pi)
PI = np.float32(np.pi)

f32 = jnp.float32
bf16 = jnp.bfloat16
i32 = jnp.int32


def _node_feat_body(x_ref, nf_ref):
    cols = []
    for b in range(B):
        xb = x_ref[b]
        px, py = xb[:, 0:1], xb[:, 1:2]
        vx, vy = xb[:, 2:3], xb[:, 3:4]
        sp = jnp.sqrt(vx * vx + vy * vy)
        inv = 1.0 / jnp.maximum(sp, 1e-30)
        c = vx * inv
        s = vy * inv
        th = jnp.arctan2(vy, vx)
        th = th + (th < 0).astype(f32) * TWO_PI
        cols += [px, py, vx, vy, c, s, th, sp]
    nf_ref[...] = jnp.concatenate(cols, axis=1)


def _edge_body(send_ref, recv_ref, recvrow_ref, nf_ref, w1_ref, b1_ref,
               sums_ref, cnts_ref):
    i = pl.program_id(0)
    nf = nf_ref[...]
    send = send_ref[...]
    recv = recv_ref[...]
    recvrow = recvrow_ref[0]

    iota_l = lax.broadcasted_iota(i32, (K, N), 1)
    oh_s = (iota_l == send).astype(f32)
    oh_r = (iota_l == recv).astype(f32)
    iota_s = lax.broadcasted_iota(i32, (N, K), 0)
    oh_rt = (iota_s == recvrow).astype(bf16)

    featj = jnp.dot(oh_s, nf, preferred_element_type=f32)
    feati = jnp.dot(oh_r, nf, preferred_element_type=f32)

    w1 = w1_ref[...]
    b1 = b1_ref[...]
    zk2 = jnp.zeros((K, 2), f32)
    zk6 = jnp.zeros((K, 6), f32)
    vals = []
    for b in range(B):
        fj = featj[:, F * b:F * b + F]
        fi = feati[:, F * b:F * b + F]
        pxj, pyj = fj[:, 0:1], fj[:, 1:2]
        vxj, vyj = fj[:, 2:3], fj[:, 3:4]
        thj = fj[:, 6:7]
        pxi, pyi = fi[:, 0:1], fi[:, 1:2]
        ci, si = fi[:, 4:5], fi[:, 5:6]
        thi = fi[:, 6:7]
        spi = fi[:, 7:8]
        dx = pxj - pxi
        dy = pyj - pyi
        rx = ci * dx + si * dy
        ry = ci * dy - si * dx
        d = thj - thi
        euler = (d + (d <= -PI).astype(f32) * TWO_PI
                 - (d > PI).astype(f32) * TWO_PI) / PI
        dist = jnp.sqrt(dx * dx + dy * dy)
        sph = jnp.arctan2(ry, rx)
        rvx = ci * vxj + si * vyj
        rvy = ci * vyj - si * vxj
        attr = jnp.concatenate(
            [rx, ry, euler, dist, sph, rvx, rvy, zk2, spi, zk6], axis=1)
        h = jnp.dot(attr, w1, preferred_element_type=f32) + b1
        v = h * (1.0 / (1.0 + jnp.exp(-h)))
        vals.append(v.astype(bf16))
    valcat = jnp.concatenate(vals, axis=1)

    part = lax.dot_general(oh_rt, valcat, (((1,), (0,)), ((), ())),
                           preferred_element_type=f32)
    cnt = lax.dot_general(oh_rt, jnp.ones((K, 8), bf16),
                          (((1,), (0,)), ((), ())),
                          preferred_element_type=f32)

    @pl.when(i == 0)
    def _():
        sums_ref[...] = part
        cnts_ref[...] = cnt

    @pl.when(i > 0)
    def _():
        sums_ref[...] += part
        cnts_ref[...] += cnt


def _final_body(sums_ref, cnts_ref, nf_ref, x_ref, w2_ref, b2_ref,
                wres_ref, bres_ref, wo1_ref, bo1_ref, wo2_ref, bo2_ref,
                wo3_ref, bo3_ref, out_ref):
    cnt = jnp.maximum(cnts_ref[...][:, 0:1], 1.0)
    agg_all = sums_ref[...] / cnt
    w2, b2 = w2_ref[...], b2_ref[...]
    wres, bres = wres_ref[...], bres_ref[...]
    wo1, bo1 = wo1_ref[...], bo1_ref[...]
    wo2, bo2 = wo2_ref[...], bo2_ref[...]
    wo3, bo3 = wo3_ref[...], bo3_ref[...]
    nf = nf_ref[...]
    for b in range(B):
        agg = agg_all[:, H * b:H * b + H]
        c = nf[:, F * b + 4:F * b + 5]
        s = nf[:, F * b + 5:F * b + 6]
        sp = nf[:, F * b + 7:F * b + 8]
        x = jnp.dot(agg, w2, preferred_element_type=f32) + b2 + sp * wres + bres
        h1 = jnp.maximum(jnp.dot(x, wo1, preferred_element_type=f32) + bo1, 0.0)
        h2 = jnp.maximum(jnp.dot(h1, wo2, preferred_element_type=f32) + bo2, 0.0)
        pred = jnp.dot(h2, wo3, preferred_element_type=f32) + bo3
        p0, p1 = pred[:, 0:1], pred[:, 1:2]
        p2, p3 = pred[:, 2:3], pred[:, 3:4]
        gx = c * p0 - s * p1
        gy = s * p0 + c * p1
        gvx = c * p2 - s * p3
        gvy = s * p2 + c * p3
        out_ref[b] = x_ref[b] + jnp.concatenate([gx, gy, gvx, gvy], axis=1)


def kernel(inputs, edges, hidden, W_ef1, b_ef1, W_ef2, b_ef2, W_res, b_res,
           W_o1, b_o1, W_o2, b_o2, W_o3, b_o3):
    del hidden
    send = edges[0].astype(i32)
    recv = edges[1].astype(i32)
    pad = EP - E
    send_p = jnp.concatenate([send, jnp.zeros((pad,), i32)])
    recv_p = jnp.concatenate([recv, jnp.full((pad,), N, i32)])
    send_col = send_p.reshape(EP, 1)
    recv_col = recv_p.reshape(EP, 1)
    recv_row = recv_p.reshape(NB, 1, K)

    nf = pl.pallas_call(
        _node_feat_body,
        out_shape=jax.ShapeDtypeStruct((N, B * F), f32),
    )(inputs)

    w1p = jnp.concatenate([W_ef1, jnp.zeros((5, H), f32)], axis=0)
    b1r = b_ef1.reshape(1, H)

    sums, cnts = pl.pallas_call(
        _edge_body,
        grid=(NB,),
        in_specs=[
            pl.BlockSpec((K, 1), lambda i: (i, 0)),
            pl.BlockSpec((K, 1), lambda i: (i, 0)),
            pl.BlockSpec((1, 1, K), lambda i: (i, 0, 0)),
            pl.BlockSpec((N, B * F), lambda i: (0, 0)),
            pl.BlockSpec((16, H), lambda i: (0, 0)),
            pl.BlockSpec((1, H), lambda i: (0, 0)),
        ],
        out_specs=[
            pl.BlockSpec((N, B * H), lambda i: (0, 0)),
            pl.BlockSpec((N, 8), lambda i: (0, 0)),
        ],
        out_shape=[
            jax.ShapeDtypeStruct((N, B * H), f32),
            jax.ShapeDtypeStruct((N, 8), f32),
        ],
    )(send_col, recv_col, recv_row, nf, w1p, b1r)

    wo3p = jnp.concatenate([W_o3, jnp.zeros((H, 4), f32)], axis=1)
    bo3p = jnp.concatenate([b_o3, jnp.zeros((4,), f32)]).reshape(1, 8)
    out = pl.pallas_call(
        _final_body,
        out_shape=jax.ShapeDtypeStruct((B, N, IN), f32),
    )(sums, cnts, nf, inputs, W_ef2, b_ef2.reshape(1, H),
      W_res[2:3, :], b_res.reshape(1, H), W_o1, b_o1.reshape(1, H),
      W_o2, b_o2.reshape(1, H), wo3p, bo3p)
    return out


# speedup vs baseline: 26.5479x; 1.6357x over previous
import functools

import jax
import jax.numpy as jnp
import numpy as np
from jax import lax
from jax.experimental import pallas as pl

B, N, E, H, IN = 4, 512, 261632, 64, 4
K = 1024
EP = 262144
NB = EP // K
F = 8
HB = H * B
TWO_PI = np.float32(2.0 * np.pi)
PI = np.float32(np.pi)

f32 = jnp.float32
bf16 = jnp.bfloat16
i32 = jnp.int32


def _node_feat_body(x_ref, nf_ref):
    feats = [[] for _ in range(F)]
    for b in range(B):
        xb = x_ref[b]
        px, py = xb[:, 0:1], xb[:, 1:2]
        vx, vy = xb[:, 2:3], xb[:, 3:4]
        sp = jnp.sqrt(vx * vx + vy * vy)
        inv = 1.0 / jnp.maximum(sp, 1e-30)
        c = vx * inv
        s = vy * inv
        th = jnp.arctan2(vy, vx)
        th = th + (th < 0).astype(f32) * TWO_PI
        for f, v in enumerate([px, py, vx, vy, c, s, th, sp]):
            feats[f].append(v)
    nf_ref[...] = jnp.concatenate([v for fl in feats for v in fl], axis=1)


def _edge_body(send_ref, recv_ref, recvrow_ref, nf_ref, w1_ref, b1_ref,
               sums_ref, cnts_ref):
    i = pl.program_id(0)
    nf = nf_ref[...]
    send = send_ref[...]
    recv = recv_ref[...]
    recvrow = recvrow_ref[0]

    iota_l = lax.broadcasted_iota(i32, (K, N), 1)
    oh_s = (iota_l == send).astype(f32)
    oh_r = (iota_l == recv).astype(f32)
    iota_s = lax.broadcasted_iota(i32, (N, K), 0)
    oh_rt = (iota_s == recvrow).astype(bf16)

    featj = jnp.dot(oh_s, nf, preferred_element_type=f32)
    feati = jnp.dot(oh_r, nf, preferred_element_type=f32)

    pxj, pyj = featj[:, 0:B], featj[:, B:2 * B]
    vxj, vyj = featj[:, 2 * B:3 * B], featj[:, 3 * B:4 * B]
    thj = featj[:, 6 * B:7 * B]
    pxi, pyi = feati[:, 0:B], feati[:, B:2 * B]
    ci, si = feati[:, 4 * B:5 * B], feati[:, 5 * B:6 * B]
    thi = feati[:, 6 * B:7 * B]
    spi = feati[:, 7 * B:8 * B]

    dx = pxj - pxi
    dy = pyj - pyi
    rx = ci * dx + si * dy
    ry = ci * dy - si * dx
    d = thj - thi
    euler = (d + (d <= -PI).astype(f32) * TWO_PI
             - (d > PI).astype(f32) * TWO_PI) / PI
    dist = jnp.sqrt(dx * dx + dy * dy)
    sph = jnp.arctan2(ry, rx)
    rvx = ci * vxj + si * vyj
    rvy = ci * vyj - si * vxj
    zb = jnp.zeros((K, B), f32)
    attr = jnp.concatenate(
        [rx, ry, euler, dist, sph, rvx, rvy, zb, zb, spi, zb, zb], axis=1)

    h = jnp.dot(attr, w1_ref[...], preferred_element_type=f32) + b1_ref[...]
    v = h * (1.0 / (1.0 + jnp.exp(-h)))
    val = v.astype(bf16)

    part = lax.dot_general(oh_rt, val, (((1,), (0,)), ((), ())),
                           preferred_element_type=f32)
    cnt = lax.dot_general(oh_rt, jnp.ones((K, 8), bf16),
                          (((1,), (0,)), ((), ())),
                          preferred_element_type=f32)

    @pl.when(i == 0)
    def _():
        sums_ref[...] = part
        cnts_ref[...] = cnt

    @pl.when(i > 0)
    def _():
        sums_ref[...] += part
        cnts_ref[...] += cnt


def _final_body(sums_ref, cnts_ref, nf_ref, x_ref, w2_ref, b2_ref,
                wres_ref, bres_ref, wo1_ref, bo1_ref, wo2_ref, bo2_ref,
                wo3_ref, bo3_ref, out_ref):
    cnt = jnp.maximum(cnts_ref[...][:, 0:1], 1.0)
    agg = sums_ref[...] / cnt
    nf = nf_ref[...]
    sp4 = nf[:, 7 * B:8 * B]
    x = (jnp.dot(agg, w2_ref[...], preferred_element_type=f32) + b2_ref[...]
         + jnp.dot(sp4, wres_ref[...], preferred_element_type=f32)
         + bres_ref[...])
    h1 = jnp.maximum(jnp.dot(x, wo1_ref[...], preferred_element_type=f32)
                     + bo1_ref[...], 0.0)
    h2 = jnp.maximum(jnp.dot(h1, wo2_ref[...], preferred_element_type=f32)
                     + bo2_ref[...], 0.0)
    pred = jnp.dot(h2, wo3_ref[...], preferred_element_type=f32) + bo3_ref[...]
    for b in range(B):
        c = nf[:, 4 * B + b:4 * B + b + 1]
        s = nf[:, 5 * B + b:5 * B + b + 1]
        p0 = pred[:, 0 * B + b:0 * B + b + 1]
        p1 = pred[:, 1 * B + b:1 * B + b + 1]
        p2 = pred[:, 2 * B + b:2 * B + b + 1]
        p3 = pred[:, 3 * B + b:3 * B + b + 1]
        gx = c * p0 - s * p1
        gy = s * p0 + c * p1
        gvx = c * p2 - s * p3
        gvy = s * p2 + c * p3
        out_ref[b] = x_ref[b] + jnp.concatenate([gx, gy, gvx, gvy], axis=1)


def _interleave(w):
    a, c = w.shape
    eye = jnp.eye(B, dtype=f32)
    return jnp.einsum('ac,bd->abcd', w, eye).reshape(a * B, c * B)


def _rep(v):
    return jnp.repeat(v, B).reshape(1, -1)


def kernel(inputs, edges, hidden, W_ef1, b_ef1, W_ef2, b_ef2, W_res, b_res,
           W_o1, b_o1, W_o2, b_o2, W_o3, b_o3):
    del hidden
    send = edges[0].astype(i32)
    recv = edges[1].astype(i32)
    pad = EP - E
    send_p = jnp.concatenate([send, jnp.zeros((pad,), i32)])
    recv_p = jnp.concatenate([recv, jnp.full((pad,), N, i32)])
    send_col = send_p.reshape(EP, 1)
    recv_col = recv_p.reshape(EP, 1)
    recv_row = recv_p.reshape(NB, 1, K)

    nf = pl.pallas_call(
        _node_feat_body,
        out_shape=jax.ShapeDtypeStruct((N, B * F), f32),
    )(inputs)

    w1p = jnp.concatenate([W_ef1, jnp.zeros((1, H), f32)], axis=0)
    w1il = _interleave(w1p)
    b1il = _rep(b_ef1)

    sums, cnts = pl.pallas_call(
        _edge_body,
        grid=(NB,),
        in_specs=[
            pl.BlockSpec((K, 1), lambda i: (i, 0)),
            pl.BlockSpec((K, 1), lambda i: (i, 0)),
            pl.BlockSpec((1, 1, K), lambda i: (i, 0, 0)),
            pl.BlockSpec((N, B * F), lambda i: (0, 0)),
            pl.BlockSpec((12 * B, HB), lambda i: (0, 0)),
            pl.BlockSpec((1, HB), lambda i: (0, 0)),
        ],
        out_specs=[
            pl.BlockSpec((N, HB), lambda i: (0, 0)),
            pl.BlockSpec((N, 8), lambda i: (0, 0)),
        ],
        out_shape=[
            jax.ShapeDtypeStruct((N, HB), f32),
            jax.ShapeDtypeStruct((N, 8), f32),
        ],
    )(send_col, recv_col, recv_row, nf, w1il, b1il)

    w2il = _interleave(W_ef2)
    wresil = _interleave(W_res[2:3, :])
    wo1il = _interleave(W_o1)
    wo2il = _interleave(W_o2)
    wo3il = _interleave(W_o3)
    out = pl.pallas_call(
        _final_body,
        out_shape=jax.ShapeDtypeStruct((B, N, IN), f32),
    )(sums, cnts, nf, inputs, w2il, _rep(b_ef2),
      wresil, _rep(b_res), wo1il, _rep(b_o1),
      wo2il, _rep(b_o2), wo3il, _rep(b_o3))
    return out


# speedup vs baseline: 27.7991x; 1.0471x over previous
import functools

import jax
import jax.numpy as jnp
import numpy as np
from jax import lax
from jax.experimental import pallas as pl

B, N, E, H, IN = 4, 512, 261632, 64, 4
K = 2048
EP = 262144
NB = EP // K
F = 8
HB = H * B
TWO_PI = np.float32(2.0 * np.pi)
PI = np.float32(np.pi)

f32 = jnp.float32
bf16 = jnp.bfloat16
i32 = jnp.int32

HALF_PI = np.float32(0.5 * np.pi)
_AT1 = np.float32(9.9997983353e-01)
_AT3 = np.float32(-3.3265547005e-01)
_AT5 = np.float32(1.9367023043e-01)
_AT7 = np.float32(-1.1665088843e-01)
_AT9 = np.float32(5.2823228825e-02)
_AT11 = np.float32(-1.1770394559e-02)


def _atan2(y, x):
    ax = jnp.abs(x)
    ay = jnp.abs(y)
    mx = jnp.maximum(ax, ay)
    mn = jnp.minimum(ax, ay)
    a = mn / jnp.maximum(mx, 1e-30)
    s = a * a
    p = ((((_AT11 * s + _AT9) * s + _AT7) * s + _AT5) * s + _AT3) * s + _AT1
    p = p * a
    p = jnp.where(ay > ax, HALF_PI - p, p)
    p = jnp.where(x < 0, PI - p, p)
    return jnp.where(y < 0, -p, p)


def _node_feat_body(x_ref, nf_ref):
    feats = [[] for _ in range(F)]
    for b in range(B):
        xb = x_ref[b]
        px, py = xb[:, 0:1], xb[:, 1:2]
        vx, vy = xb[:, 2:3], xb[:, 3:4]
        sp = jnp.sqrt(vx * vx + vy * vy)
        inv = 1.0 / jnp.maximum(sp, 1e-30)
        c = vx * inv
        s = vy * inv
        th = _atan2(vy, vx)
        th = th + (th < 0).astype(f32) * TWO_PI
        for f, v in enumerate([px, py, vx, vy, c, s, th, sp]):
            feats[f].append(v)
    nf_ref[...] = jnp.concatenate([v for fl in feats for v in fl], axis=1)


def _edge_body(send_ref, recv_ref, recvrow_ref, nf_ref, w1_ref, b1_ref,
               sums_ref, cnts_ref):
    i = pl.program_id(0)
    nf = nf_ref[...]
    send = send_ref[...]
    recv = recv_ref[...]
    recvrow = recvrow_ref[0]

    iota_l = lax.broadcasted_iota(i32, (K, N), 1)
    oh_s = (iota_l == send).astype(bf16)
    oh_r = (iota_l == recv).astype(bf16)
    iota_s = lax.broadcasted_iota(i32, (N, K), 0)
    oh_rt = (iota_s == recvrow).astype(bf16)

    nfb = nf.astype(bf16)
    featj = jnp.dot(oh_s, nfb, preferred_element_type=f32)
    feati = jnp.dot(oh_r, nfb, preferred_element_type=f32)

    pxj, pyj = featj[:, 0:B], featj[:, B:2 * B]
    vxj, vyj = featj[:, 2 * B:3 * B], featj[:, 3 * B:4 * B]
    thj = featj[:, 6 * B:7 * B]
    pxi, pyi = feati[:, 0:B], feati[:, B:2 * B]
    ci, si = feati[:, 4 * B:5 * B], feati[:, 5 * B:6 * B]
    thi = feati[:, 6 * B:7 * B]
    spi = feati[:, 7 * B:8 * B]

    dx = pxj - pxi
    dy = pyj - pyi
    rx = ci * dx + si * dy
    ry = ci * dy - si * dx
    d = thj - thi
    euler = (d + (d <= -PI).astype(f32) * TWO_PI
             - (d > PI).astype(f32) * TWO_PI) / PI
    dist = jnp.sqrt(dx * dx + dy * dy)
    sph = _atan2(ry, rx)
    rvx = ci * vxj + si * vyj
    rvy = ci * vyj - si * vxj
    zb = jnp.zeros((K, B), f32)
    attr = jnp.concatenate(
        [rx, ry, euler, dist, sph, rvx, rvy, zb, zb, spi, zb, zb], axis=1)

    h = jnp.dot(attr, w1_ref[...], preferred_element_type=f32) + b1_ref[...]
    v = h * (1.0 / (1.0 + jnp.exp(-h)))
    val = v.astype(bf16)

    part = lax.dot_general(oh_rt, val, (((1,), (0,)), ((), ())),
                           preferred_element_type=f32)
    cnt = lax.dot_general(oh_rt, jnp.ones((K, 8), bf16),
                          (((1,), (0,)), ((), ())),
                          preferred_element_type=f32)

    @pl.when(i == 0)
    def _():
        sums_ref[...] = part
        cnts_ref[...] = cnt

    @pl.when(i > 0)
    def _():
        sums_ref[...] += part
        cnts_ref[...] += cnt


def _final_body(sums_ref, cnts_ref, nf_ref, x_ref, w2_ref, b2_ref,
                wres_ref, bres_ref, wo1_ref, bo1_ref, wo2_ref, bo2_ref,
                wo3_ref, bo3_ref, out_ref):
    cnt = jnp.maximum(cnts_ref[...][:, 0:1], 1.0)
    agg = sums_ref[...] / cnt
    nf = nf_ref[...]
    sp4 = nf[:, 7 * B:8 * B]
    x = (jnp.dot(agg, w2_ref[...], preferred_element_type=f32) + b2_ref[...]
         + jnp.dot(sp4, wres_ref[...], preferred_element_type=f32)
         + bres_ref[...])
    h1 = jnp.maximum(jnp.dot(x, wo1_ref[...], preferred_element_type=f32)
                     + bo1_ref[...], 0.0)
    h2 = jnp.maximum(jnp.dot(h1, wo2_ref[...], preferred_element_type=f32)
                     + bo2_ref[...], 0.0)
    pred = jnp.dot(h2, wo3_ref[...], preferred_element_type=f32) + bo3_ref[...]
    for b in range(B):
        c = nf[:, 4 * B + b:4 * B + b + 1]
        s = nf[:, 5 * B + b:5 * B + b + 1]
        p0 = pred[:, 0 * B + b:0 * B + b + 1]
        p1 = pred[:, 1 * B + b:1 * B + b + 1]
        p2 = pred[:, 2 * B + b:2 * B + b + 1]
        p3 = pred[:, 3 * B + b:3 * B + b + 1]
        gx = c * p0 - s * p1
        gy = s * p0 + c * p1
        gvx = c * p2 - s * p3
        gvy = s * p2 + c * p3
        out_ref[b] = x_ref[b] + jnp.concatenate([gx, gy, gvx, gvy], axis=1)


def _interleave(w):
    a, c = w.shape
    eye = jnp.eye(B, dtype=f32)
    return jnp.einsum('ac,bd->abcd', w, eye).reshape(a * B, c * B)


def _rep(v):
    return jnp.repeat(v, B).reshape(1, -1)


def kernel(inputs, edges, hidden, W_ef1, b_ef1, W_ef2, b_ef2, W_res, b_res,
           W_o1, b_o1, W_o2, b_o2, W_o3, b_o3):
    del hidden
    send = edges[0].astype(i32)
    recv = edges[1].astype(i32)
    pad = EP - E
    send_p = jnp.concatenate([send, jnp.zeros((pad,), i32)])
    recv_p = jnp.concatenate([recv, jnp.full((pad,), N, i32)])
    send_col = send_p.reshape(EP, 1)
    recv_col = recv_p.reshape(EP, 1)
    recv_row = recv_p.reshape(NB, 1, K)

    nf = pl.pallas_call(
        _node_feat_body,
        out_shape=jax.ShapeDtypeStruct((N, B * F), f32),
    )(inputs)

    w1p = jnp.concatenate([W_ef1, jnp.zeros((1, H), f32)], axis=0)
    w1il = _interleave(w1p)
    b1il = _rep(b_ef1)

    sums, cnts = pl.pallas_call(
        _edge_body,
        grid=(NB,),
        in_specs=[
            pl.BlockSpec((K, 1), lambda i: (i, 0)),
            pl.BlockSpec((K, 1), lambda i: (i, 0)),
            pl.BlockSpec((1, 1, K), lambda i: (i, 0, 0)),
            pl.BlockSpec((N, B * F), lambda i: (0, 0)),
            pl.BlockSpec((12 * B, HB), lambda i: (0, 0)),
            pl.BlockSpec((1, HB), lambda i: (0, 0)),
        ],
        out_specs=[
            pl.BlockSpec((N, HB), lambda i: (0, 0)),
            pl.BlockSpec((N, 8), lambda i: (0, 0)),
        ],
        out_shape=[
            jax.ShapeDtypeStruct((N, HB), f32),
            jax.ShapeDtypeStruct((N, 8), f32),
        ],
    )(send_col, recv_col, recv_row, nf, w1il, b1il)

    w2il = _interleave(W_ef2)
    wresil = _interleave(W_res[2:3, :])
    wo1il = _interleave(W_o1)
    wo2il = _interleave(W_o2)
    wo3il = _interleave(W_o3)
    out = pl.pallas_call(
        _final_body,
        out_shape=jax.ShapeDtypeStruct((B, N, IN), f32),
    )(sums, cnts, nf, inputs, w2il, _rep(b_ef2),
      wresil, _rep(b_res), wo1il, _rep(b_o1),
      wo2il, _rep(b_o2), wo3il, _rep(b_o3))
    return out


# speedup vs baseline: 53.1795x; 1.9130x over previous
import functools

import jax
import jax.numpy as jnp
import numpy as np
from jax import lax
from jax.experimental import pallas as pl

B, N, E, H, IN = 4, 512, 261632, 64, 4
N2 = 2 * N
K = 2048
EP = 262144
NB = EP // K
HB = H * B
TWO_PI = np.float32(2.0 * np.pi)
PI = np.float32(np.pi)

f32 = jnp.float32
bf16 = jnp.bfloat16
i32 = jnp.int32

HALF_PI = np.float32(0.5 * np.pi)
_AT1 = np.float32(9.9997983353e-01)
_AT3 = np.float32(-3.3265547005e-01)
_AT5 = np.float32(1.9367023043e-01)
_AT7 = np.float32(-1.1665088843e-01)
_AT9 = np.float32(5.2823228825e-02)
_AT11 = np.float32(-1.1770394559e-02)


def _atan2(y, x):
    ax = jnp.abs(x)
    ay = jnp.abs(y)
    mx = jnp.maximum(ax, ay)
    mn = jnp.minimum(ax, ay)
    a = mn / jnp.maximum(mx, 1e-30)
    s = a * a
    p = ((((_AT11 * s + _AT9) * s + _AT7) * s + _AT5) * s + _AT3) * s + _AT1
    p = p * a
    p = jnp.where(ay > ax, HALF_PI - p, p)
    p = jnp.where(x < 0, PI - p, p)
    return jnp.where(y < 0, -p, p)


def _node_feat_body(x_ref, sel_ref, nf_ref):
    pxs, pys, vxs, vys, cs, ss, ths, sps = [], [], [], [], [], [], [], []
    for b in range(B):
        xt = jnp.transpose(x_ref[b])
        px, py = xt[0:1], xt[1:2]
        vx, vy = xt[2:3], xt[3:4]
        sp = jnp.sqrt(vx * vx + vy * vy)
        inv = 1.0 / jnp.maximum(sp, 1e-30)
        c = vx * inv
        s = vy * inv
        th = _atan2(vy, vx)
        th = th + (th < 0).astype(f32) * TWO_PI
        pxs.append(px)
        pys.append(py)
        vxs.append(vx)
        vys.append(vy)
        cs.append(c)
        ss.append(s)
        ths.append(th)
        sps.append(sp)

    z = jnp.zeros((1, N), f32)
    rows = []
    for grp in (cs, ss, cs, ss, cs, ss, cs, ss):
        for b in range(B):
            rows.append(jnp.concatenate([z, grp[b]], axis=1))
    for grp, neg in ((pxs, True), (pys, True), (pys, True), (pxs, True),
                     (vxs, False), (vys, False), (vys, False), (vxs, False)):
        for b in range(B):
            rows.append(jnp.concatenate(
                [grp[b], -grp[b] if neg else z], axis=1))
    z2 = jnp.concatenate([z, z], axis=1)
    for grp, kind in ((pxs, 'd'), (pys, 'd'), (ths, 'd'), (sps, 'i')):
        for b in range(B):
            if kind == 'd':
                rows.append(jnp.concatenate([grp[b], -grp[b]], axis=1))
            else:
                rows.append(jnp.concatenate([z, grp[b]], axis=1))
        rows.extend([z2] * B)
    sel_ref[...] = jnp.concatenate(rows, axis=0).astype(bf16)

    nfr = []
    for grp in (cs, ss, sps):
        nfr.extend([grp[b] for b in range(B)])
        nfr.extend([z] * B)
    nf_ref[...] = jnp.concatenate(nfr, axis=0)


def _edge_body(sendrow_ref, recvnrow_ref, recv_ref, sel_ref, w1p_ref,
               w1n_ref, cr_ref, b1_ref, sums_ref, cnts_ref):
    i = pl.program_id(0)
    sendr = sendrow_ref[0]
    recvnr = recvnrow_ref[0]
    recv = recv_ref[...]

    iota2 = lax.broadcasted_iota(i32, (N2, K), 0)
    oht = ((iota2 == sendr) | (iota2 == recvnr)).astype(bf16)

    g = jnp.dot(sel_ref[...], oht, preferred_element_type=f32)
    l1 = g[0:32]
    l2 = g[32:64]
    p = l1 * l2
    rt = jnp.dot(cr_ref[...], p, preferred_element_type=f32)
    rxs, rys = rt[0:4], rt[8:12]
    dxs, dys = g[64:68], g[72:76]
    ds, spis = g[80:84], g[88:92]

    euler = (ds + (ds <= -PI).astype(f32) * TWO_PI
             - (ds > PI).astype(f32) * TWO_PI) / PI
    dist = jnp.sqrt(dxs * dxs + dys * dys)
    sph = _atan2(rys, rxs)
    nl = jnp.concatenate([euler, dist, sph, spis], axis=0)

    ht = (jnp.dot(w1p_ref[...], p.astype(bf16), preferred_element_type=f32)
          + jnp.dot(w1n_ref[...], nl.astype(bf16), preferred_element_type=f32)
          + b1_ref[...])
    vt = ht * (1.0 / (1.0 + jnp.exp(-ht)))
    val = vt.astype(bf16)

    iota_l = lax.broadcasted_iota(i32, (K, N), 1)
    oh_r = (iota_l == recv).astype(bf16)

    part = jnp.dot(val, oh_r, preferred_element_type=f32)
    cnt = jnp.dot(jnp.ones((8, K), bf16), oh_r,
                  preferred_element_type=f32)

    @pl.when(i == 0)
    def _():
        sums_ref[...] = part
        cnts_ref[...] = cnt

    @pl.when(i > 0)
    def _():
        sums_ref[...] += part
        cnts_ref[...] += cnt


def _final_body(sums_ref, cnts_ref, nf_ref, xin_ref, w2_ref, b2_ref,
                wres_ref, bres_ref, wo1_ref, bo1_ref, wo2_ref, bo2_ref,
                wo3_ref, bo3_ref, out_ref):
    cnt = jnp.maximum(cnts_ref[...][0:1], 1.0)
    agg = sums_ref[...] / cnt
    nf = nf_ref[...]
    sp4 = nf[16:20]
    x = (jnp.dot(w2_ref[...], agg, preferred_element_type=f32) + b2_ref[...]
         + jnp.dot(wres_ref[...], sp4, preferred_element_type=f32)
         + bres_ref[...])
    h1 = jnp.maximum(jnp.dot(wo1_ref[...], x, preferred_element_type=f32)
                     + bo1_ref[...], 0.0)
    h2 = jnp.maximum(jnp.dot(wo2_ref[...], h1, preferred_element_type=f32)
                     + bo2_ref[...], 0.0)
    pred = jnp.dot(wo3_ref[...], h2, preferred_element_type=f32) + bo3_ref[...]
    p0, p1 = pred[0:4], pred[8:12]
    p2, p3 = pred[16:20], pred[24:28]
    c, s = nf[0:4], nf[8:12]
    gx = c * p0 - s * p1
    gy = s * p0 + c * p1
    gvx = c * p2 - s * p3
    gvy = s * p2 + c * p3
    out_ref[...] = xin_ref[...] + jnp.concatenate([gx, gy, gvx, gvy], axis=0)


def _interleave(w):
    a, c = w.shape
    eye = jnp.eye(B, dtype=f32)
    return jnp.einsum('ac,bd->abcd', w, eye).reshape(a * B, c * B)


def _colrep(v):
    return jnp.repeat(v, B).reshape(-1, 1)


_CR = np.zeros((16, 32), np.float32)
for _b in range(4):
    _CR[_b, 0 + _b] = 1.0
    _CR[_b, 4 + _b] = 1.0
    _CR[8 + _b, 8 + _b] = 1.0
    _CR[8 + _b, 12 + _b] = -1.0
del _b


def kernel(inputs, edges, hidden, W_ef1, b_ef1, W_ef2, b_ef2, W_res, b_res,
           W_o1, b_o1, W_o2, b_o2, W_o3, b_o3):
    del hidden
    send = edges[0].astype(i32)
    recv = edges[1].astype(i32)
    pad = EP - E
    send_p = jnp.concatenate([send, jnp.zeros((pad,), i32)])
    recv_p = jnp.concatenate([recv, jnp.full((pad,), N, i32)])
    send_row = send_p.reshape(NB, 1, K)
    recvn_row = (recv_p + N).reshape(NB, 1, K)
    recv_col = recv_p.reshape(EP, 1)

    sel, nf = pl.pallas_call(
        _node_feat_body,
        out_shape=[
            jax.ShapeDtypeStruct((96, N2), bf16),
            jax.ShapeDtypeStruct((24, N), f32),
        ],
    )(inputs)

    w1 = W_ef1
    w1p_small = jnp.stack([w1[0], w1[0], w1[1], -w1[1],
                           w1[5], w1[5], w1[6], -w1[6]])
    w1pt = _interleave(w1p_small).T.astype(bf16)
    w1n_small = jnp.stack([w1[2], w1[3], w1[4], w1[9]])
    w1nt = _interleave(w1n_small).T.astype(bf16)
    b1t = _colrep(b_ef1)

    sums, cnts = pl.pallas_call(
        _edge_body,
        grid=(NB,),
        in_specs=[
            pl.BlockSpec((1, 1, K), lambda i: (i, 0, 0)),
            pl.BlockSpec((1, 1, K), lambda i: (i, 0, 0)),
            pl.BlockSpec((K, 1), lambda i: (i, 0)),
            pl.BlockSpec((96, N2), lambda i: (0, 0)),
            pl.BlockSpec((HB, 32), lambda i: (0, 0)),
            pl.BlockSpec((HB, 16), lambda i: (0, 0)),
            pl.BlockSpec((16, 32), lambda i: (0, 0)),
            pl.BlockSpec((HB, 1), lambda i: (0, 0)),
        ],
        out_specs=[
            pl.BlockSpec((HB, N), lambda i: (0, 0)),
            pl.BlockSpec((8, N), lambda i: (0, 0)),
        ],
        out_shape=[
            jax.ShapeDtypeStruct((HB, N), f32),
            jax.ShapeDtypeStruct((8, N), f32),
        ],
    )(send_row, recvn_row, recv_col, sel, w1pt, w1nt,
      jnp.asarray(_CR), b1t)

    w2t = _interleave(W_ef2).T
    wrest = _interleave(W_res[2:3, :]).T
    wo1t = _interleave(W_o1).T
    wo2t = _interleave(W_o2).T
    wo3t = jnp.zeros((32, H, B), f32)
    for o in range(IN):
        for b in range(B):
            wo3t = wo3t.at[o * 8 + b, :, b].set(W_o3[:, o])
    wo3t = wo3t.reshape(32, HB)
    bo3t = jnp.zeros((32, 1), f32)
    for o in range(IN):
        bo3t = bo3t.at[o * 8:o * 8 + B, 0].set(b_o3[o])

    xin_t = jnp.transpose(inputs, (2, 0, 1)).reshape(16, N)
    out16 = pl.pallas_call(
        _final_body,
        out_shape=jax.ShapeDtypeStruct((16, N), f32),
    )(sums, cnts, nf, xin_t, w2t, _colrep(b_ef2),
      wrest, _colrep(b_res), wo1t, _colrep(b_o1),
      wo2t, _colrep(b_o2), wo3t, bo3t)
    return out16.reshape(IN, B, N).transpose(1, 2, 0)
